# baseline (speedup 1.0000x reference)
import functools

import jax
import jax.numpy as jnp
import numpy as np
from jax import lax
from jax.experimental import pallas as pl
from jax.experimental.pallas import tpu as pltpu
from jax.experimental.pallas import tpu_sc as plsc

NC = 2
NS = 16
NW = NC * NS

CHUNK_ROWS = 512


@functools.partial(jax.jit, static_argnames=("batch", "seq", "embed"))
def _sc_gather_pool(x_flat, table, dst_pat, zeros, *, batch, seq, embed):
    elems_per_w = batch // NW
    rows_per_w = elems_per_w * seq
    nchunks = rows_per_w // CHUNK_ROWS
    assert nchunks % 2 == 0
    mesh = plsc.VectorSubcoreMesh(core_axis_name="c", subcore_axis_name="s")

    @functools.partial(
        pl.kernel,
        out_type=jax.ShapeDtypeStruct((batch, embed), jnp.float32),
        mesh=mesh,
        compiler_params=pltpu.CompilerParams(use_tc_tiling_on_sc=False),
        scratch_types=[
            pltpu.VMEM((rows_per_w,), jnp.int32),
            pltpu.VMEM((CHUNK_ROWS, embed), jnp.float32),
            pltpu.VMEM((CHUNK_ROWS, embed), jnp.float32),
            pltpu.VMEM((nchunks, CHUNK_ROWS), jnp.int32),
            pltpu.VMEM_SHARED((NS * elems_per_w, embed), jnp.float32),
            pltpu.SemaphoreType.DMA,
            pltpu.SemaphoreType.DMA,
        ],
    )
    def k(x_hbm, table_hbm, dstpat_hbm, zeros_hbm, out_hbm,
          idx_v, buf0, buf1, dst_v, acc_sh, sem0, sem1):
        c = lax.axis_index("c")
        s = lax.axis_index("s")
        wid = s * NC + c
        row_base = wid * rows_per_w
        bufs = (buf0, buf1)
        sems = (sem0, sem1)

        pltpu.sync_copy(x_hbm.at[pl.ds(row_base, rows_per_w)], idx_v)
        pltpu.sync_copy(dstpat_hbm.at[wid], dst_v)
        pltpu.sync_copy(zeros_hbm, acc_sh.at[pl.ds(s * elems_per_w, elems_per_w)])

        def start_gather(i, b):
            pltpu.async_copy(
                table_hbm.at[idx_v.at[pl.ds(i * CHUNK_ROWS, CHUNK_ROWS)]],
                bufs[b], sems[b])

        def pool(i, b):
            pltpu.sync_copy(bufs[b], acc_sh.at[dst_v.at[i]], add=True)

        start_gather(0, 0)

        def pair(g, _):
            i0 = g * 2
            pltpu.make_async_copy(table_hbm, buf0, sem0).wait()
            start_gather(i0 + 1, 1)
            pool(i0, 0)
            pltpu.make_async_copy(table_hbm, buf1, sem1).wait()
            start_gather(i0 + 2, 0)
            pool(i0 + 1, 1)
            return ()

        lax.fori_loop(0, nchunks // 2 - 1, pair, (), unroll=False)

        pltpu.make_async_copy(table_hbm, buf0, sem0).wait()
        start_gather(nchunks - 1, 1)
        pool(nchunks - 2, 0)
        pltpu.make_async_copy(table_hbm, buf1, sem1).wait()
        pool(nchunks - 1, 1)

        pltpu.sync_copy(acc_sh.at[pl.ds(s * elems_per_w, elems_per_w)],
                        out_hbm.at[pl.ds(wid * elems_per_w, elems_per_w)])

    return k(x_flat, table, dst_pat, zeros)


def _dense_softmax(pooled_sum, W, b2, inv_seq, block_b):
    batch, embed = pooled_sum.shape
    out = W.shape[1]

    def body(p_ref, w_ref, b_ref, o_ref):
        logits = jnp.dot(p_ref[...] * inv_seq, w_ref[...],
                         preferred_element_type=jnp.float32) + b_ref[...]
        m = jnp.max(logits, axis=-1, keepdims=True)
        e = jnp.exp(logits - m)
        o_ref[...] = e / jnp.sum(e, axis=-1, keepdims=True)

    return pl.pallas_call(
        body,
        grid=(batch // block_b,),
        in_specs=[
            pl.BlockSpec((block_b, embed), lambda i: (i, 0)),
            pl.BlockSpec((embed, out), lambda i: (0, 0)),
            pl.BlockSpec((1, out), lambda i: (0, 0)),
        ],
        out_specs=pl.BlockSpec((block_b, out), lambda i: (i, 0)),
        out_shape=jax.ShapeDtypeStruct((batch, out), jnp.float32),
    )(pooled_sum, W, b2)


def kernel(x, table, W, b):
    batch, seq = x.shape
    vocab, embed = table.shape
    elems_per_w = batch // NW

    e_idx = np.repeat(np.arange(elems_per_w, dtype=np.int32), seq)
    dst_pat = (e_idx[None, :] +
               (np.arange(NW, dtype=np.int32)[:, None] // NC) * elems_per_w)
    dst_pat = dst_pat.reshape(NW, -1, CHUNK_ROWS).astype(np.int32)
    zeros = jnp.zeros((elems_per_w, embed), jnp.float32)

    pooled_sum = _sc_gather_pool(x.reshape(-1), table, jnp.asarray(dst_pat),
                                 zeros, batch=batch, seq=seq, embed=embed)
    return _dense_softmax(pooled_sum, W, b.reshape(1, -1), 1.0 / seq, 256)
